# baseline (speedup 1.0000x reference)
import functools

import jax
import jax.numpy as jnp
from jax import lax
from jax.experimental import pallas as pl
from jax.experimental.pallas import tpu as pltpu
from jax.experimental.pallas import tpu_sc as plsc

VOCAB = 128
SEQ = 72
D = 1024
DH = D // 2
BATCH = 4096

NC = 2
NS = 16
NW = NC * NS
LANES = 16

NTOK = BATCH * SEQ
TOK_PER_W = NTOK // NW
CHUNK = 16
N_CHUNKS = TOK_PER_W // CHUNK
S_BLK = 8

NG = 4
STEADY_LO = 2
STEADY_N = (N_CHUNKS - 6 - STEADY_LO) // 4
assert STEADY_LO + 4 * STEADY_N == N_CHUNKS - 6
assert CHUNK == LANES


def _shuffle_pairs(a):
    n = a.shape[-1]
    return (
        a.reshape(a.shape[:-1] + (n // 32, 2, 16))
        .swapaxes(-2, -1)
        .reshape(a.shape[:-1] + (n,))
    )


def _fused_body(tok_ref, pos_ref, out_ref):
    s = tok_ref[...][None, :, :] + pos_ref[...][:, None, :]
    out_ref[...] = s.astype(jnp.bfloat16)


def _build_fused(tok_embed, pos2d):
    return pl.pallas_call(
        _fused_body,
        grid=(SEQ // S_BLK,),
        in_specs=[
            pl.BlockSpec((VOCAB, D), lambda s: (0, 0)),
            pl.BlockSpec((S_BLK, D), lambda s: (s, 0)),
        ],
        out_specs=pl.BlockSpec((S_BLK, VOCAB, D), lambda s: (s, 0, 0)),
        out_shape=jax.ShapeDtypeStruct((SEQ, VOCAB, D), jnp.bfloat16),
    )(tok_embed, pos2d)


_MESH = plsc.VectorSubcoreMesh(core_axis_name="c", subcore_axis_name="s")


@functools.partial(
    pl.kernel,
    out_type=jax.ShapeDtypeStruct((NTOK, D), jnp.int32),
    mesh=_MESH,
    scratch_types=[
        [pltpu.VMEM((CHUNK, DH), jnp.int32) for _ in range(NG)],
        [pltpu.VMEM((CHUNK, D), jnp.int32) for _ in range(2)],
        pltpu.VMEM((NG, CHUNK), jnp.int32),
        [pltpu.SemaphoreType.DMA for _ in range(NG)],
        [pltpu.SemaphoreType.DMA for _ in range(NG)],
        [pltpu.SemaphoreType.DMA for _ in range(2)],
    ],
)
def _sc_kernel(idx_hbm, fused_hbm, out_hbm,
               rowsbf, outb, idxr, gsems, isems, ssems):
    cid = lax.axis_index("c")
    sid = lax.axis_index("s")
    wid = sid * NC + cid
    base = wid * TOK_PER_W

    def fire_idx(j, sl):
        pltpu.async_copy(idx_hbm.at[pl.ds((base // CHUNK + j) * CHUNK, CHUNK)],
                         idxr.at[sl], isems[sl])

    def wait_idx(j, sl):
        pltpu.make_async_copy(
            idx_hbm.at[pl.ds((base // CHUNK + j) * CHUNK, CHUNK)],
            idxr.at[sl], isems[sl]).wait()

    def fuse(j, sl):
        p = base + j * CHUNK + lax.iota(jnp.int32, LANES)
        idxr[sl, :] = lax.rem(p, SEQ) * VOCAB + idxr[sl, :]

    def fire_g(j, sl):
        pltpu.async_copy(fused_hbm.at[idxr.at[sl]], rowsbf[sl], gsems[sl])

    def wait_g(j, sl):
        pltpu.make_async_copy(fused_hbm.at[idxr.at[sl]], rowsbf[sl],
                              gsems[sl]).wait()

    def fire_s(i, sl):
        pltpu.async_copy(outb[sl], out_hbm.at[pl.ds(base + i * CHUNK, CHUNK)],
                         ssems[sl])

    def wait_s(i, sl):
        pltpu.make_async_copy(outb[sl],
                              out_hbm.at[pl.ds(base + i * CHUNK, CHUNK)],
                              ssems[sl]).wait()

    sixteen = jnp.full((LANES,), 16, jnp.int32)
    mask = jnp.full((LANES,), -65536, jnp.int32)

    def conv2(bg, bo):
        def per_tok(t, carry):
            for c in range(DH // LANES):
                w = rowsbf[bg][t, pl.ds(c * LANES, LANES)]
                outb[bo][t, pl.ds(2 * c * LANES, LANES)] = (
                    lax.shift_left(w, sixteen))
                outb[bo][t, pl.ds((2 * c + 1) * LANES, LANES)] = (
                    lax.bitwise_and(w, mask))
            return carry

        lax.fori_loop(0, CHUNK, per_tok, 0)

    def pipe_iter(i, jm, do_ws=True, do_g=True, do_fi=True):
        bg = jm % NG
        bg3 = (jm + 3) % NG
        bo = jm % 2
        wait_g(i, bg)
        if do_fi:
            fire_idx(i + NG, bg)
        if do_g:
            wait_idx(i + 3, bg3)
            fuse(i + 3, bg3)
            fire_g(i + 3, bg3)
        if do_ws:
            wait_s(i - 2, bo)
        conv2(bg, bo)
        fire_s(i, bo)

    for sl in range(NG):
        fire_idx(sl, sl)
    for j in range(3):
        wait_idx(j, j)
        fuse(j, j)
        fire_g(j, j)
    for i in range(STEADY_LO):
        pipe_iter(i, i, do_ws=False)

    def step(k, carry):
        for jj in range(4):
            pipe_iter(STEADY_LO + k * 4 + jj, STEADY_LO + jj)
        return carry

    lax.fori_loop(0, STEADY_N, step, 0)

    for i in range(N_CHUNKS - 6, N_CHUNKS):
        pipe_iter(i, i,
                  do_g=i + 3 <= N_CHUNKS - 1,
                  do_fi=i + NG <= N_CHUNKS - 1)
    wait_s(N_CHUNKS - 2, (N_CHUNKS - 2) % 2)
    wait_s(N_CHUNKS - 1, (N_CHUNKS - 1) % 2)


def kernel(x, tok_embed, pos_embed):
    tok_s = _shuffle_pairs(tok_embed.astype(jnp.float32))
    pos_s = _shuffle_pairs(pos_embed.reshape(SEQ, D).astype(jnp.float32))
    fused_bf = _build_fused(tok_s, pos_s)
    fused_i32 = lax.bitcast_convert_type(
        fused_bf.reshape(SEQ * VOCAB, DH, 2), jnp.int32)
    x1d = x.reshape(NTOK).astype(jnp.int32)
    out = _sc_kernel(x1d, fused_i32)
    return lax.bitcast_convert_type(out, jnp.float32).reshape(BATCH, SEQ, D)


# speedup vs baseline: 1.4329x; 1.4329x over previous
import functools

import jax
import jax.numpy as jnp
from jax import lax
from jax.experimental import pallas as pl
from jax.experimental.pallas import tpu as pltpu
from jax.experimental.pallas import tpu_sc as plsc

VOCAB = 128
SEQ = 72
D = 1024
DH = D // 2
BATCH = 4096

NC = 2
NS = 16
NW = NC * NS
LANES = 16

NTOK = BATCH * SEQ
TOK_PER_W = NTOK // NW
CHUNK = 16
N_CHUNKS = TOK_PER_W // CHUNK
S_BLK = 8

NG = 4
STEADY_LO = 2
STEADY_N = (N_CHUNKS - 6 - STEADY_LO) // 4
assert STEADY_LO + 4 * STEADY_N == N_CHUNKS - 6
assert CHUNK == LANES


def _shuffle_pairs(a):
    n = a.shape[-1]
    return (
        a.reshape(a.shape[:-1] + (n // 32, 2, 16))
        .swapaxes(-2, -1)
        .reshape(a.shape[:-1] + (n,))
    )


def _fused_body(tok_ref, pos_ref, out_ref):
    s = tok_ref[...][None, :, :] + pos_ref[...][:, None, :]
    out_ref[...] = s.astype(jnp.bfloat16)


def _build_fused(tok_embed, pos2d):
    return pl.pallas_call(
        _fused_body,
        grid=(SEQ // S_BLK,),
        in_specs=[
            pl.BlockSpec((VOCAB, D), lambda s: (0, 0)),
            pl.BlockSpec((S_BLK, D), lambda s: (s, 0)),
        ],
        out_specs=pl.BlockSpec((S_BLK, VOCAB, D), lambda s: (s, 0, 0)),
        out_shape=jax.ShapeDtypeStruct((SEQ, VOCAB, D), jnp.bfloat16),
    )(tok_embed, pos2d)


_MESH = plsc.VectorSubcoreMesh(core_axis_name="c", subcore_axis_name="s")


@functools.partial(
    pl.kernel,
    out_type=jax.ShapeDtypeStruct((NTOK, D), jnp.int32),
    mesh=_MESH,
    scratch_types=[
        [pltpu.VMEM((CHUNK, DH), jnp.int32) for _ in range(NG)],
        [pltpu.VMEM((CHUNK, D), jnp.int32) for _ in range(2)],
        pltpu.VMEM((NG, CHUNK), jnp.int32),
        [pltpu.SemaphoreType.DMA for _ in range(NG)],
        [pltpu.SemaphoreType.DMA for _ in range(NG)],
        [pltpu.SemaphoreType.DMA for _ in range(2)],
    ],
)
def _sc_kernel(idx_hbm, fused_hbm, out_hbm,
               rowsbf, outb, idxr, gsems, isems, ssems):
    cid = lax.axis_index("c")
    sid = lax.axis_index("s")
    wid = sid * NC + cid
    base = wid * TOK_PER_W

    def fire_idx(j, sl):
        pltpu.async_copy(idx_hbm.at[pl.ds((base // CHUNK + j) * CHUNK, CHUNK)],
                         idxr.at[sl], isems[sl])

    def wait_idx(j, sl):
        pltpu.make_async_copy(
            idx_hbm.at[pl.ds((base // CHUNK + j) * CHUNK, CHUNK)],
            idxr.at[sl], isems[sl]).wait()

    def fuse(j, sl):
        p = base + j * CHUNK + lax.iota(jnp.int32, LANES)
        idxr[sl, :] = lax.rem(p, SEQ) * VOCAB + idxr[sl, :]

    def fire_g(j, sl):
        pltpu.async_copy(fused_hbm.at[idxr.at[sl]], rowsbf[sl], gsems[sl])

    def wait_g(j, sl):
        pltpu.make_async_copy(fused_hbm.at[idxr.at[sl]], rowsbf[sl],
                              gsems[sl]).wait()

    def fire_s(i, sl):
        pltpu.async_copy(outb[sl], out_hbm.at[pl.ds(base + i * CHUNK, CHUNK)],
                         ssems[sl])

    def wait_s(i, sl):
        pltpu.make_async_copy(outb[sl],
                              out_hbm.at[pl.ds(base + i * CHUNK, CHUNK)],
                              ssems[sl]).wait()

    sixteen = jnp.full((LANES,), 16, jnp.int32)
    mask = jnp.full((LANES,), -65536, jnp.int32)

    def conv2(bg, bo):
        @plsc.parallel_loop(0, CHUNK * DH // LANES, unroll=8)
        def _(u):
            t = lax.shift_right_logical(u, 5)
            c16 = pl.multiple_of(
                lax.shift_left(lax.bitwise_and(u, 31), 4), LANES)
            o32 = pl.multiple_of(2 * c16, LANES)
            w = rowsbf[bg][t, pl.ds(c16, LANES)]
            outb[bo][t, pl.ds(o32, LANES)] = lax.shift_left(w, sixteen)
            outb[bo][t, pl.ds(o32 + LANES, LANES)] = (
                lax.bitwise_and(w, mask))

    def pipe_iter(i, jm, do_ws=True, do_g=True, do_fi=True):
        bg = jm % NG
        bg3 = (jm + 3) % NG
        bo = jm % 2
        wait_g(i, bg)
        if do_fi:
            fire_idx(i + NG, bg)
        if do_g:
            wait_idx(i + 3, bg3)
            fuse(i + 3, bg3)
            fire_g(i + 3, bg3)
        if do_ws:
            wait_s(i - 2, bo)
        conv2(bg, bo)
        fire_s(i, bo)

    for sl in range(NG):
        fire_idx(sl, sl)
    for j in range(3):
        wait_idx(j, j)
        fuse(j, j)
        fire_g(j, j)
    for i in range(STEADY_LO):
        pipe_iter(i, i, do_ws=False)

    def step(k, carry):
        for jj in range(4):
            pipe_iter(STEADY_LO + k * 4 + jj, STEADY_LO + jj)
        return carry

    lax.fori_loop(0, STEADY_N, step, 0)

    for i in range(N_CHUNKS - 6, N_CHUNKS):
        pipe_iter(i, i,
                  do_g=i + 3 <= N_CHUNKS - 1,
                  do_fi=i + NG <= N_CHUNKS - 1)
    wait_s(N_CHUNKS - 2, (N_CHUNKS - 2) % 2)
    wait_s(N_CHUNKS - 1, (N_CHUNKS - 1) % 2)


def kernel(x, tok_embed, pos_embed):
    tok_s = _shuffle_pairs(tok_embed.astype(jnp.float32))
    pos_s = _shuffle_pairs(pos_embed.reshape(SEQ, D).astype(jnp.float32))
    fused_bf = _build_fused(tok_s, pos_s)
    fused_i32 = lax.bitcast_convert_type(
        fused_bf.reshape(SEQ * VOCAB, DH, 2), jnp.int32)
    x1d = x.reshape(NTOK).astype(jnp.int32)
    out = _sc_kernel(x1d, fused_i32)
    return lax.bitcast_convert_type(out, jnp.float32).reshape(BATCH, SEQ, D)


# speedup vs baseline: 1.4330x; 1.0001x over previous
import functools

import jax
import jax.numpy as jnp
from jax import lax
from jax.experimental import pallas as pl
from jax.experimental.pallas import tpu as pltpu
from jax.experimental.pallas import tpu_sc as plsc

VOCAB = 128
SEQ = 72
D = 1024
DH = D // 2
BATCH = 4096

NC = 2
NS = 16
NW = NC * NS
LANES = 16

NTOK = BATCH * SEQ
TOK_PER_W = NTOK // NW
CHUNK = 16
N_CHUNKS = TOK_PER_W // CHUNK
S_BLK = 8

NG = 4
STEADY_LO = 2
STEADY_N = (N_CHUNKS - 6 - STEADY_LO) // 4
assert STEADY_LO + 4 * STEADY_N == N_CHUNKS - 6
assert CHUNK == LANES


def _shuffle_pairs(a):
    n = a.shape[-1]
    return (
        a.reshape(a.shape[:-1] + (n // 32, 2, 16))
        .swapaxes(-2, -1)
        .reshape(a.shape[:-1] + (n,))
    )


def _fused_body(tok_ref, pos_ref, out_ref):
    s = tok_ref[...][None, :, :] + pos_ref[...][:, None, :]
    out_ref[...] = s.astype(jnp.bfloat16)


def _build_fused(tok_embed, pos2d):
    return pl.pallas_call(
        _fused_body,
        grid=(SEQ // S_BLK,),
        in_specs=[
            pl.BlockSpec((VOCAB, D), lambda s: (0, 0)),
            pl.BlockSpec((S_BLK, D), lambda s: (s, 0)),
        ],
        out_specs=pl.BlockSpec((S_BLK, VOCAB, D), lambda s: (s, 0, 0)),
        out_shape=jax.ShapeDtypeStruct((SEQ, VOCAB, D), jnp.bfloat16),
    )(tok_embed, pos2d)


_MESH = plsc.VectorSubcoreMesh(core_axis_name="c", subcore_axis_name="s")


@functools.partial(
    pl.kernel,
    out_type=jax.ShapeDtypeStruct((NTOK, D), jnp.int32),
    mesh=_MESH,
    scratch_types=[
        [pltpu.VMEM((CHUNK, DH), jnp.int32) for _ in range(NG)],
        [pltpu.VMEM((CHUNK, D), jnp.int32) for _ in range(2)],
        pltpu.VMEM((NG, CHUNK), jnp.int32),
        [pltpu.SemaphoreType.DMA for _ in range(NG)],
        [pltpu.SemaphoreType.DMA for _ in range(NG)],
        [pltpu.SemaphoreType.DMA for _ in range(2)],
    ],
)
def _sc_kernel(idx_hbm, fused_hbm, out_hbm,
               rowsbf, outb, idxr, gsems, isems, ssems):
    cid = lax.axis_index("c")
    sid = lax.axis_index("s")
    wid = sid * NC + cid
    base = wid * TOK_PER_W

    def fire_idx(j, sl):
        pltpu.async_copy(idx_hbm.at[pl.ds((base // CHUNK + j) * CHUNK, CHUNK)],
                         idxr.at[sl], isems[sl])

    def wait_idx(j, sl):
        pltpu.make_async_copy(
            idx_hbm.at[pl.ds((base // CHUNK + j) * CHUNK, CHUNK)],
            idxr.at[sl], isems[sl]).wait()

    def fuse(j, sl):
        p = base + j * CHUNK + lax.iota(jnp.int32, LANES)
        idxr[sl, :] = lax.rem(p, SEQ) * VOCAB + idxr[sl, :]

    def fire_g(j, sl):
        pltpu.async_copy(fused_hbm.at[idxr.at[sl]], rowsbf[sl], gsems[sl])

    def wait_g(j, sl):
        pltpu.make_async_copy(fused_hbm.at[idxr.at[sl]], rowsbf[sl],
                              gsems[sl]).wait()

    def fire_s(i, sl):
        pltpu.async_copy(outb[sl], out_hbm.at[pl.ds(base + i * CHUNK, CHUNK)],
                         ssems[sl])

    def wait_s(i, sl):
        pltpu.make_async_copy(outb[sl],
                              out_hbm.at[pl.ds(base + i * CHUNK, CHUNK)],
                              ssems[sl]).wait()

    sixteen = jnp.full((LANES,), 16, jnp.int32)
    mask = jnp.full((LANES,), -65536, jnp.int32)

    def conv2(bg, bo):
        def per_tok(t, carry):
            @plsc.parallel_loop(0, DH // LANES, unroll=8)
            def _(c):
                c16 = pl.multiple_of(lax.shift_left(c, 4), LANES)
                o32 = pl.multiple_of(lax.shift_left(c, 5), LANES)
                w = rowsbf[bg][t, pl.ds(c16, LANES)]
                outb[bo][t, pl.ds(o32, LANES)] = lax.shift_left(w, sixteen)
                outb[bo][t, pl.ds(o32 + LANES, LANES)] = (
                    lax.bitwise_and(w, mask))

            return carry

        lax.fori_loop(0, CHUNK, per_tok, 0)

    def pipe_iter(i, jm, do_ws=True, do_g=True, do_fi=True):
        bg = jm % NG
        bg3 = (jm + 3) % NG
        bo = jm % 2
        wait_g(i, bg)
        if do_fi:
            fire_idx(i + NG, bg)
        if do_g:
            wait_idx(i + 3, bg3)
            fuse(i + 3, bg3)
            fire_g(i + 3, bg3)
        if do_ws:
            wait_s(i - 2, bo)
        conv2(bg, bo)
        fire_s(i, bo)

    for sl in range(NG):
        fire_idx(sl, sl)
    for j in range(3):
        wait_idx(j, j)
        fuse(j, j)
        fire_g(j, j)
    for i in range(STEADY_LO):
        pipe_iter(i, i, do_ws=False)

    def step(k, carry):
        for jj in range(4):
            pipe_iter(STEADY_LO + k * 4 + jj, STEADY_LO + jj)
        return carry

    lax.fori_loop(0, STEADY_N, step, 0)

    for i in range(N_CHUNKS - 6, N_CHUNKS):
        pipe_iter(i, i,
                  do_g=i + 3 <= N_CHUNKS - 1,
                  do_fi=i + NG <= N_CHUNKS - 1)
    wait_s(N_CHUNKS - 2, (N_CHUNKS - 2) % 2)
    wait_s(N_CHUNKS - 1, (N_CHUNKS - 1) % 2)


def kernel(x, tok_embed, pos_embed):
    tok_s = _shuffle_pairs(tok_embed.astype(jnp.float32))
    pos_s = _shuffle_pairs(pos_embed.reshape(SEQ, D).astype(jnp.float32))
    fused_bf = _build_fused(tok_s, pos_s)
    fused_i32 = lax.bitcast_convert_type(
        fused_bf.reshape(SEQ * VOCAB, DH, 2), jnp.int32)
    x1d = x.reshape(NTOK).astype(jnp.int32)
    out = _sc_kernel(x1d, fused_i32)
    return lax.bitcast_convert_type(out, jnp.float32).reshape(BATCH, SEQ, D)


# speedup vs baseline: 2.6745x; 1.8664x over previous
import functools

import jax
import jax.numpy as jnp
from jax import lax
from jax.experimental import pallas as pl
from jax.experimental.pallas import tpu as pltpu
from jax.experimental.pallas import tpu_sc as plsc

VOCAB = 128
SEQ = 72
D = 1024
BATCH = 4096

NC = 2
NS = 16
NW = NC * NS
LANES = 16

NTOK = BATCH * SEQ
TOK_PER_W = NTOK // NW
CHUNK = 16
N_CHUNKS = TOK_PER_W // CHUNK
NBUF = 3
assert CHUNK % LANES == 0 and TOK_PER_W % CHUNK == 0 and (N_CHUNKS - 3) % NBUF == 0


S_BLK = 8


def _fused_body(tok_ref, pos_ref, out_ref):
    out_ref[...] = tok_ref[...][None, :, :] + pos_ref[...][:, None, :]


def _build_fused(tok_embed, pos2d):
    return pl.pallas_call(
        _fused_body,
        grid=(SEQ // S_BLK,),
        in_specs=[
            pl.BlockSpec((VOCAB, D), lambda s: (0, 0)),
            pl.BlockSpec((S_BLK, D), lambda s: (s, 0)),
        ],
        out_specs=pl.BlockSpec((S_BLK, VOCAB, D), lambda s: (s, 0, 0)),
        out_shape=jax.ShapeDtypeStruct((SEQ, VOCAB, D), jnp.float32),
    )(tok_embed, pos2d)


_MESH = plsc.VectorSubcoreMesh(core_axis_name="c", subcore_axis_name="s")


@functools.partial(
    pl.kernel,
    out_type=jax.ShapeDtypeStruct((NTOK, D), jnp.float32),
    mesh=_MESH,
    scratch_types=[
        pltpu.VMEM((N_CHUNKS, CHUNK), jnp.int32),
        [pltpu.VMEM((CHUNK, D), jnp.float32) for _ in range(NBUF)],
        [pltpu.SemaphoreType.DMA for _ in range(NBUF)],
        [pltpu.SemaphoreType.DMA for _ in range(NBUF)],
    ],
)
def _gather_kernel(idx_hbm, fused_hbm, out_hbm, idx_v, rows, gsems, ssems):
    wid = lax.axis_index("s") * NC + lax.axis_index("c")
    base = wid * TOK_PER_W

    pltpu.sync_copy(idx_hbm.at[pl.ds(wid * N_CHUNKS, N_CHUNKS)], idx_v)

    def fuse(c, carry):
        for l in range(CHUNK // LANES):
            p = base + c * CHUNK + l * LANES + lax.iota(jnp.int32, LANES)
            sl = (c, pl.ds(l * LANES, LANES))
            idx_v[sl] = lax.rem(p, SEQ) * VOCAB + idx_v[sl]
        return carry

    lax.fori_loop(0, N_CHUNKS, fuse, 0)

    def fire_g(i, b):
        pltpu.async_copy(fused_hbm.at[idx_v.at[i]], rows[b], gsems[b])

    def wait_g(i, b):
        pltpu.make_async_copy(fused_hbm.at[idx_v.at[i]], rows[b], gsems[b]).wait()

    def fire_s(i, b):
        pltpu.async_copy(rows[b], out_hbm.at[pl.ds(base + i * CHUNK, CHUNK)], ssems[b])

    def wait_s(i, b):
        pltpu.make_async_copy(
            rows[b], out_hbm.at[pl.ds(base + i * CHUNK, CHUNK)], ssems[b]
        ).wait()

    fire_g(0, 0)
    fire_g(1, 1)
    wait_g(0, 0)
    fire_s(0, 0)
    fire_g(2, 2)

    def step(k, carry):
        for b in range(NBUF):
            i = k * NBUF + b + 1
            bi = (b + 1) % NBUF
            wait_g(i, bi)
            fire_s(i, bi)
            wait_s(i - 1, b)
            fire_g(i + 2, b)
        return carry

    lax.fori_loop(0, (N_CHUNKS - 3) // NBUF, step, 0)

    for i in (N_CHUNKS - 2, N_CHUNKS - 1):
        wait_g(i, i % NBUF)
        fire_s(i, i % NBUF)
    for i in (N_CHUNKS - 3, N_CHUNKS - 2, N_CHUNKS - 1):
        wait_s(i, i % NBUF)


def kernel(x, tok_embed, pos_embed):
    pos2d = pos_embed.reshape(SEQ, D).astype(jnp.float32)
    fused = _build_fused(tok_embed.astype(jnp.float32), pos2d)
    fused_flat = fused.reshape(SEQ * VOCAB, D)
    x2d = x.reshape(NW * N_CHUNKS, CHUNK).astype(jnp.int32)
    out = _gather_kernel(x2d, fused_flat)
    return out.reshape(BATCH, SEQ, D)
